# Initial kernel scaffold; baseline (speedup 1.0000x reference)
import jax
import jax.numpy as jnp
from jax.experimental import pallas as pl


def kernel(x, pos_emb, gamma, beta):
    raise NotImplementedError("write your pallas kernel here")


# speedup vs baseline: 1.3428x; 1.3428x over previous
import functools

import jax
import jax.numpy as jnp
from jax.experimental import pallas as pl


def _ln_kernel(x_ref, pe_ref, g_ref, b_ref, o_ref, *, chunk):
    c = pl.program_id(1)
    v = x_ref[0] + pe_ref[:, pl.ds(c * chunk, chunk)]
    inv_d = 1.0 / v.shape[0]
    mean = jnp.sum(v, axis=0, keepdims=True) * inv_d
    var = jnp.sum(v * v, axis=0, keepdims=True) * inv_d - mean * mean
    rstd = jax.lax.rsqrt(var + 1e-5)
    o_ref[0] = (v - mean) * rstd * g_ref[...] + b_ref[...]


def kernel(x, pos_emb, gamma, beta):
    b, dim, lp = x.shape
    chunk = 1024
    n_c = lp // chunk
    pe_t = pos_emb.T
    g = gamma.reshape(dim, 1)
    bt = beta.reshape(dim, 1)
    return pl.pallas_call(
        functools.partial(_ln_kernel, chunk=chunk),
        grid=(b, n_c),
        in_specs=[
            pl.BlockSpec((1, dim, chunk), lambda i, j: (i, 0, j)),
            pl.BlockSpec((dim, lp), lambda i, j: (0, 0)),
            pl.BlockSpec((dim, 1), lambda i, j: (0, 0)),
            pl.BlockSpec((dim, 1), lambda i, j: (0, 0)),
        ],
        out_specs=pl.BlockSpec((1, dim, chunk), lambda i, j: (i, 0, j)),
        out_shape=jax.ShapeDtypeStruct((b, dim, lp), x.dtype),
    )(x, pe_t, g, bt)


# speedup vs baseline: 2.1255x; 1.5829x over previous
import jax
import jax.numpy as jnp
from jax.experimental import pallas as pl


def _ln_kernel(x_ref, pe_ref, g_ref, b_ref, o_ref):
    v = x_ref[0] + pe_ref[...]
    inv_d = 1.0 / v.shape[0]
    mean = jnp.sum(v, axis=0, keepdims=True) * inv_d
    var = jnp.sum(v * v, axis=0, keepdims=True) * inv_d - mean * mean
    rstd = jax.lax.rsqrt(var + 1e-5)
    o_ref[0] = (v - mean) * rstd * g_ref[...] + b_ref[...]


def kernel(x, pos_emb, gamma, beta):
    b, dim, lp = x.shape
    pe_t = pos_emb.T
    g = gamma.reshape(dim, 1)
    bt = beta.reshape(dim, 1)
    return pl.pallas_call(
        _ln_kernel,
        grid=(b,),
        in_specs=[
            pl.BlockSpec((1, dim, lp), lambda i: (i, 0, 0)),
            pl.BlockSpec((dim, lp), lambda i: (0, 0)),
            pl.BlockSpec((dim, 1), lambda i: (0, 0)),
            pl.BlockSpec((dim, 1), lambda i: (0, 0)),
        ],
        out_specs=pl.BlockSpec((1, dim, lp), lambda i: (i, 0, 0)),
        out_shape=jax.ShapeDtypeStruct((b, dim, lp), x.dtype),
    )(x, pe_t, g, bt)


# speedup vs baseline: 2.3616x; 1.1111x over previous
import jax
import jax.numpy as jnp
from jax.experimental import pallas as pl


def _ln_kernel(x_ref, pe_ref, g_ref, b_ref, o_ref):
    rows, lp = x_ref.shape
    dim = pe_ref.shape[0]
    nb = rows // dim
    v = x_ref[...].reshape(nb, dim, lp) + pe_ref[...][None]
    inv_d = 1.0 / dim
    mean = jnp.sum(v, axis=1, keepdims=True) * inv_d
    var = jnp.sum(v * v, axis=1, keepdims=True) * inv_d - mean * mean
    rstd = jax.lax.rsqrt(var + 1e-5)
    o = (v - mean) * rstd * g_ref[...][None] + b_ref[...][None]
    o_ref[...] = o.reshape(rows, lp)


def kernel(x, pos_emb, gamma, beta):
    b, dim, lp = x.shape
    xf = x.reshape(b * dim, lp)
    rows = 512
    pe_t = pos_emb.T
    g = gamma.reshape(dim, 1)
    bt = beta.reshape(dim, 1)
    out = pl.pallas_call(
        _ln_kernel,
        grid=(b * dim // rows,),
        in_specs=[
            pl.BlockSpec((rows, lp), lambda i: (i, 0)),
            pl.BlockSpec((dim, lp), lambda i: (0, 0)),
            pl.BlockSpec((dim, 1), lambda i: (0, 0)),
            pl.BlockSpec((dim, 1), lambda i: (0, 0)),
        ],
        out_specs=pl.BlockSpec((rows, lp), lambda i: (i, 0)),
        out_shape=jax.ShapeDtypeStruct((b * dim, lp), x.dtype),
    )(xf, pe_t, g, bt)
    return out.reshape(b, dim, lp)
